# baseline (speedup 1.0000x reference)
import jax
import jax.numpy as jnp
from jax.experimental import pallas as pl
from jax.experimental.pallas import tpu as pltpu

_ROWS = 100000
_COLS = 512
_CHUNK = 2000
_NCH = _ROWS // _CHUNK


def _body(x_hbm, o_hbm):
    def inner(buf, in_sem0, in_sem1, out_sem0, out_sem1):
        in_sems = (in_sem0, in_sem1)
        out_sems = (out_sem0, out_sem1)

        def start_in(j):
            return pltpu.async_copy(
                x_hbm.at[pl.ds(j * _CHUNK, _CHUNK), :],
                buf.at[j & 1],
                in_sems[j & 1],
            )

        def start_out(j):
            return pltpu.async_copy(
                buf.at[j & 1],
                o_hbm.at[pl.ds(j * _CHUNK, _CHUNK), :],
                out_sems[j & 1],
            )

        ins = {0: start_in(0)}
        outs = {}
        for j in range(_NCH):
            ins.pop(j).wait()
            if j + 1 < _NCH:
                if j >= 1:
                    outs.pop(j - 1).wait()
                ins[j + 1] = start_in(j + 1)
            if j == 0:
                two = jnp.full((1, _COLS), 2.0, jnp.float32)
                buf[0, pl.ds(1, 1), :] = two
                buf[0, pl.ds(3, 1), :] = two
            outs[j] = start_out(j)
        outs.pop(_NCH - 2).wait()
        outs.pop(_NCH - 1).wait()

    pl.run_scoped(
        inner,
        pltpu.VMEM((2, _CHUNK, _COLS), jnp.float32),
        pltpu.SemaphoreType.DMA,
        pltpu.SemaphoreType.DMA,
        pltpu.SemaphoreType.DMA,
        pltpu.SemaphoreType.DMA,
    )


_tc_mesh = pltpu.create_tensorcore_mesh("tc")

_copy_kernel = pl.kernel(
    _body,
    mesh=_tc_mesh,
    out_type=jax.ShapeDtypeStruct((_ROWS, _COLS), jnp.float32),
)


def kernel(x):
    return _copy_kernel(x)


# speedup vs baseline: 1.4781x; 1.4781x over previous
import jax
import jax.numpy as jnp
from jax.experimental import pallas as pl

_ROWS = 100000
_COLS = 512
_BLOCK = 4000


def _body(x_ref, o_ref):
    o_ref[...] = x_ref[...]

    @pl.when(pl.program_id(0) == 0)
    def _overwrite():
        two = jnp.full((1, _COLS), 2.0, jnp.float32)
        o_ref[pl.ds(1, 1), :] = two
        o_ref[pl.ds(3, 1), :] = two


def kernel(x):
    return pl.pallas_call(
        _body,
        grid=(_ROWS // _BLOCK,),
        in_specs=[pl.BlockSpec((_BLOCK, _COLS), lambda i: (i, 0))],
        out_specs=pl.BlockSpec((_BLOCK, _COLS), lambda i: (i, 0)),
        out_shape=jax.ShapeDtypeStruct((_ROWS, _COLS), jnp.float32),
    )(x)


# speedup vs baseline: 1.4784x; 1.0001x over previous
import jax
import jax.numpy as jnp
from jax.experimental import pallas as pl

_ROWS = 100000
_COLS = 512
_BLOCK = 5000


def _body(x_ref, o_ref):
    o_ref[...] = x_ref[...]

    @pl.when(pl.program_id(0) == 0)
    def _overwrite():
        two = jnp.full((1, _COLS), 2.0, jnp.float32)
        o_ref[pl.ds(1, 1), :] = two
        o_ref[pl.ds(3, 1), :] = two


def kernel(x):
    return pl.pallas_call(
        _body,
        grid=(_ROWS // _BLOCK,),
        in_specs=[pl.BlockSpec((_BLOCK, _COLS), lambda i: (i, 0))],
        out_specs=pl.BlockSpec((_BLOCK, _COLS), lambda i: (i, 0)),
        out_shape=jax.ShapeDtypeStruct((_ROWS, _COLS), jnp.float32),
    )(x)


# speedup vs baseline: 1.4808x; 1.0017x over previous
import jax
import jax.numpy as jnp
from jax.experimental import pallas as pl
from jax.experimental.pallas import tpu as pltpu

_ROWS = 100000
_COLS = 512
_BLOCK = 5000


def _body(x_ref, o_ref):
    o_ref[...] = x_ref[...]

    @pl.when(pl.program_id(0) == 0)
    def _overwrite():
        two = jnp.full((1, _COLS), 2.0, jnp.float32)
        o_ref[pl.ds(1, 1), :] = two
        o_ref[pl.ds(3, 1), :] = two


def kernel(x):
    return pl.pallas_call(
        _body,
        grid=(_ROWS // _BLOCK,),
        in_specs=[pl.BlockSpec((_BLOCK, _COLS), lambda i: (i, 0))],
        out_specs=pl.BlockSpec((_BLOCK, _COLS), lambda i: (i, 0)),
        out_shape=jax.ShapeDtypeStruct((_ROWS, _COLS), jnp.float32),
        compiler_params=pltpu.CompilerParams(
            dimension_semantics=("parallel",),
        ),
    )(x)
